# baseline (speedup 1.0000x reference)
import functools
import math

import jax
import jax.numpy as jnp
import numpy as np
from jax import lax
from jax.experimental import pallas as pl
from jax.experimental.pallas import tpu as pltpu
from jax.experimental.pallas import tpu_sc as plsc

_VOCAB = 1000000
_EMB = 64
_BATCH = 1024
_SEQ = 200

_NC, _NS = 2, 16
_NW = _NC * _NS
_N = _BATCH * _SEQ
_K = 128
_ROWS_PER_W = _N // _NW
_CHUNKS = _ROWS_PER_W // _K
_VREGS = _EMB // 16
_LBLK = _BATCH // _K


def _pe_table():
    position = np.arange(0, _SEQ, dtype=np.float64)[:, None]
    div_term = np.exp(
        np.arange(0, _EMB, 2, dtype=np.float64) * -(math.log(10000.0) / _EMB))
    pe = np.zeros((_SEQ, _EMB), dtype=np.float32)
    pe[:, 0::2] = np.sin(position * div_term).astype(np.float32)
    pe[:, 1::2] = np.cos(position * div_term).astype(np.float32)
    return pe


_PE = _pe_table()


def _body(idx_hbm, pe_hbm, table_hbm, out_hbm, idx_v, pe_v, buf_v, sem):
    w = lax.axis_index("s") * _NC + lax.axis_index("c")
    pltpu.sync_copy(idx_hbm.at[w], idx_v)
    pltpu.sync_copy(pe_hbm, pe_v)
    row0 = w * _ROWS_PER_W

    def chunk(c, carry):
        g = w * _CHUNKS + c
        l = g // _LBLK
        pltpu.async_copy(table_hbm.at[idx_v.at[c]], buf_v, sem).wait()
        pe_regs = [pe_v[l, pl.ds(16 * j, 16)] for j in range(_VREGS)]

        def row(r, rcarry):
            for j in range(_VREGS):
                buf_v[r, pl.ds(16 * j, 16)] += pe_regs[j]
            return rcarry

        lax.fori_loop(0, _K, row, 0)
        pltpu.sync_copy(buf_v, out_hbm.at[pl.ds(row0 + c * _K, _K)])
        return carry

    lax.fori_loop(0, _CHUNKS, chunk, 0)


_sc_call = functools.partial(
    pl.kernel,
    out_type=jax.ShapeDtypeStruct((_N, _EMB), jnp.float32),
    mesh=plsc.VectorSubcoreMesh(
        core_axis_name="c", subcore_axis_name="s",
        num_cores=_NC, num_subcores=_NS),
    scratch_types=[
        pltpu.VMEM((_CHUNKS, _K), jnp.int32),
        pltpu.VMEM((_SEQ, _EMB), jnp.float32),
        pltpu.VMEM((_K, _EMB), jnp.float32),
        pltpu.SemaphoreType.DMA,
    ],
    compiler_params=pltpu.CompilerParams(use_tc_tiling_on_sc=False),
)(_body)


def kernel(input, table):
    idx = input.T.reshape(_NW, _CHUNKS, _K)
    out = _sc_call(idx, _PE, table)
    return out.reshape(_SEQ, _BATCH, _EMB)


# speedup vs baseline: 1.0866x; 1.0866x over previous
import functools
import math

import jax
import jax.numpy as jnp
import numpy as np
from jax import lax
from jax.experimental import pallas as pl
from jax.experimental.pallas import tpu as pltpu
from jax.experimental.pallas import tpu_sc as plsc

_VOCAB = 1000000
_EMB = 64
_BATCH = 1024
_SEQ = 200

_NC, _NS = 2, 16
_NW = _NC * _NS
_N = _BATCH * _SEQ
_K = 128
_ROWS_PER_W = _N // _NW
_CHUNKS = _ROWS_PER_W // _K
_VREGS = _EMB // 16
_LBLK = _BATCH // _K
_RING = 5
_GPC = _CHUNKS // _RING


def _pe_table():
    position = np.arange(0, _SEQ, dtype=np.float64)[:, None]
    div_term = np.exp(
        np.arange(0, _EMB, 2, dtype=np.float64) * -(math.log(10000.0) / _EMB))
    pe = np.zeros((_SEQ, _EMB), dtype=np.float32)
    pe[:, 0::2] = np.sin(position * div_term).astype(np.float32)
    pe[:, 1::2] = np.cos(position * div_term).astype(np.float32)
    return pe


_PE = _pe_table()


def _body(idx_hbm, pe_hbm, table_hbm, out_hbm, idx_v, pe_v, gbuf, obuf, *sems):
    gsem, osem = sems[:_RING], sems[_RING:]
    w = lax.axis_index("s") * _NC + lax.axis_index("c")
    pltpu.sync_copy(idx_hbm.at[w], idx_v)
    pltpu.sync_copy(pe_hbm, pe_v)
    row0 = w * _ROWS_PER_W

    def gather(c, k):
        return pltpu.make_async_copy(
            table_hbm.at[idx_v.at[c]], gbuf.at[k], gsem[k])

    def scatter(c, k):
        return pltpu.make_async_copy(
            obuf.at[k], out_hbm.at[pl.ds(row0 + c * _K, _K)], osem[k])

    for k in range(_RING):
        gather(k, k).start()

    def turn(i, carry):
        for k in range(_RING):
            c = i * _RING + k
            gather(c, k).wait()

            @pl.when(i > 0)
            def _():
                scatter(c - _RING, k).wait()

            l = (w * _CHUNKS + c) // _LBLK
            pe_regs = [pe_v[l, pl.ds(16 * j, 16)] for j in range(_VREGS)]
            gb, ob = gbuf.at[k], obuf.at[k]

            def _rows(r8, rcarry):
                for u in range(8):
                    r = r8 * 8 + u
                    for j in range(_VREGS):
                        ob[r, pl.ds(16 * j, 16)] = (
                            gb[r, pl.ds(16 * j, 16)] + pe_regs[j])
                return rcarry

            lax.fori_loop(0, _K // 8, _rows, 0)

            @pl.when(i < _GPC - 1)
            def _():
                gather(c + _RING, k).start()

            scatter(c, k).start()
        return carry

    lax.fori_loop(0, _GPC, turn, 0)

    for k in range(_RING):
        scatter((_GPC - 1) * _RING + k, k).wait()


_sc_call = functools.partial(
    pl.kernel,
    out_type=jax.ShapeDtypeStruct((_N, _EMB), jnp.float32),
    mesh=plsc.VectorSubcoreMesh(
        core_axis_name="c", subcore_axis_name="s",
        num_cores=_NC, num_subcores=_NS),
    scratch_types=[
        pltpu.VMEM((_CHUNKS, _K), jnp.int32),
        pltpu.VMEM((_SEQ, _EMB), jnp.float32),
        pltpu.VMEM((_RING, _K, _EMB), jnp.float32),
        pltpu.VMEM((_RING, _K, _EMB), jnp.float32),
    ] + [pltpu.SemaphoreType.DMA] * (2 * _RING),
    compiler_params=pltpu.CompilerParams(use_tc_tiling_on_sc=False),
)(_body)


def kernel(input, table):
    idx = input.T.reshape(_NW, _CHUNKS, _K)
    out = _sc_call(idx, _PE, table)
    return out.reshape(_SEQ, _BATCH, _EMB)
